# Initial kernel scaffold; baseline (speedup 1.0000x reference)
import jax
import jax.numpy as jnp
from jax.experimental import pallas as pl


def kernel(adj_list, feat, attention_weights, gat_W, gat_al, gat_ar, gat_b, gm_W, gm_al, gm_ar, gm_b, sem_W1, sem_b1, sem_q, ft_W, ft_b):
    raise NotImplementedError("write your pallas kernel here")


# speedup vs baseline: 1.5147x; 1.5147x over previous
import jax
import jax.numpy as jnp
from jax import lax
from jax.experimental import pallas as pl

_N = 1024
_HID = 64
_M = 2
_SEM_HID = 128
_BJ = 256


def _gat_one(cnt, h, h_blk, el, ar, b):
    er_blk = lax.dot_general(ar, h_blk, (((1,), (1,)), ((), ())),
                             preferred_element_type=jnp.float32)
    e = jax.nn.leaky_relu(el + er_blk, negative_slope=0.2)
    has = cnt > 0.0
    emax = jnp.max(jnp.where(has, e, -jnp.inf), axis=0, keepdims=True)
    ee = jnp.where(has, jnp.exp(e - emax), 0.0) * cnt
    denom = jnp.sum(ee, axis=0, keepdims=True)
    alpha = ee / denom
    out = lax.dot_general(alpha, h, (((0,), (0,)), ((), ())),
                          preferred_element_type=jnp.float32)
    return jnp.tanh(out + b)


def _gat3_kernel(adj_ref, feat_ref, aw_ref, gat_W_ref, gat_al_ref,
                 gat_ar_ref, gat_b_ref, gm_W_ref, gm_al_ref, gm_ar_ref,
                 gm_b_ref, mg_ref, m0_ref, m1_ref):
    j = pl.program_id(0)
    feat = feat_ref[...]
    feat_blk = feat_ref[pl.ds(j * _BJ, _BJ), :]

    a0 = adj_ref[0, :, :]
    a1 = adj_ref[1, :, :]
    a0f = a0.astype(jnp.float32)
    a1f = a1.astype(jnp.float32)

    row = lax.broadcasted_iota(jnp.int32, (_N, _BJ), 0)
    col = lax.broadcasted_iota(jnp.int32, (_N, _BJ), 1) + j * _BJ
    eye = (row == col).astype(jnp.float32)

    w = jax.nn.softmax(aw_ref[...])
    merged = a0f * w[0:1, 0:1] + a1f * w[0:1, 1:2]
    cnt_m = (merged != 0.0).astype(jnp.float32) + eye
    cnt_0 = (a0 != 0).astype(jnp.float32) + eye
    cnt_1 = (a1 != 0).astype(jnp.float32) + eye

    def run(cnt, W, al, ar, b):
        h = jnp.dot(feat, W, preferred_element_type=jnp.float32)
        h_blk = jnp.dot(feat_blk, W, preferred_element_type=jnp.float32)
        el = jnp.sum(h * al, axis=1, keepdims=True)
        return _gat_one(cnt, h, h_blk, el, ar, b)

    mg_ref[...] = run(cnt_m, gat_W_ref[...], gat_al_ref[...],
                      gat_ar_ref[...], gat_b_ref[...])
    m0_ref[...] = run(cnt_0, gm_W_ref[0], gm_al_ref[0:1, :],
                      gm_ar_ref[0:1, :], gm_b_ref[0:1, :])
    m1_ref[...] = run(cnt_1, gm_W_ref[1], gm_al_ref[1:2, :],
                      gm_ar_ref[1:2, :], gm_b_ref[1:2, :])


def _combine_kernel(mg_ref, m0_ref, m1_ref, sem_W1_ref, sem_b1_ref,
                    sem_q_ref, ft_W_ref, ft_b_ref, out_ref):
    mg = mg_ref[...]
    m0 = m0_ref[...]
    m1 = m1_ref[...]
    sem_W1 = sem_W1_ref[...]
    sem_b1 = sem_b1_ref[...]
    sem_q = sem_q_ref[...]

    def wp(x):
        t = jnp.tanh(jnp.dot(x, sem_W1, preferred_element_type=jnp.float32)
                     + sem_b1)
        return jnp.dot(t, sem_q, preferred_element_type=jnp.float32)

    s0 = jnp.sum(wp(mg)) / _N
    s1 = jnp.sum(wp(m0)) / _N
    s2 = jnp.sum(wp(m1)) / _N
    smax = jnp.maximum(jnp.maximum(s0, s1), s2)
    e0 = jnp.exp(s0 - smax)
    e1 = jnp.exp(s1 - smax)
    e2 = jnp.exp(s2 - smax)
    tot = e0 + e1 + e2
    semantic = (e0 / tot) * mg + (e1 / tot) * m0 + (e2 / tot) * m1

    ft_W = ft_W_ref[...]
    fa = (jnp.dot(mg, ft_W[0:_HID, :], preferred_element_type=jnp.float32)
          + jnp.dot(semantic, ft_W[_HID:2 * _HID, :],
                    preferred_element_type=jnp.float32)
          + ft_b_ref[...])
    out_ref[...] = jnp.tanh(fa)


def kernel(adj_list, feat, attention_weights, gat_W, gat_al, gat_ar, gat_b,
           gm_W, gm_al, gm_ar, gm_b, sem_W1, sem_b1, sem_q, ft_W, ft_b):
    aw = attention_weights.reshape(1, _M)
    gat_al2 = gat_al.reshape(1, _HID)
    gat_ar2 = gat_ar.reshape(1, _HID)
    gat_b2 = gat_b.reshape(1, _HID)
    sem_b12 = sem_b1.reshape(1, _SEM_HID)
    sem_q2 = sem_q.reshape(_SEM_HID, 1)
    ft_b2 = ft_b.reshape(1, _HID)

    num_blocks = _N // _BJ
    full = lambda shape: pl.BlockSpec(shape, lambda j: (0,) * len(shape))
    mg, m0, m1 = pl.pallas_call(
        _gat3_kernel,
        grid=(num_blocks,),
        in_specs=[
            pl.BlockSpec((_M, _N, _BJ), lambda j: (0, 0, j)),
            full((_N, _HID)),
            full((1, _M)),
            full((_HID, _HID)),
            full((1, _HID)),
            full((1, _HID)),
            full((1, _HID)),
            full((_M, _HID, _HID)),
            full((_M, _HID)),
            full((_M, _HID)),
            full((_M, _HID)),
        ],
        out_specs=[
            pl.BlockSpec((_BJ, _HID), lambda j: (j, 0)),
            pl.BlockSpec((_BJ, _HID), lambda j: (j, 0)),
            pl.BlockSpec((_BJ, _HID), lambda j: (j, 0)),
        ],
        out_shape=[jax.ShapeDtypeStruct((_N, _HID), jnp.float32)] * 3,
    )(adj_list, feat, aw, gat_W, gat_al2, gat_ar2, gat_b2,
      gm_W, gm_al, gm_ar, gm_b)

    out = pl.pallas_call(
        _combine_kernel,
        out_shape=jax.ShapeDtypeStruct((_N, _HID), jnp.float32),
    )(mg, m0, m1, sem_W1, sem_b12, sem_q2, ft_W, ft_b2)
    return out


# speedup vs baseline: 1.5229x; 1.0054x over previous
import jax
import jax.numpy as jnp
from jax import lax
from jax.experimental import pallas as pl

_N = 1024
_HID = 64
_M = 2
_SEM_HID = 128
_BJ = 256
_SLOPE = 0.2


def _gat_one(cnt_f, feat, feat_blk, W, al, ar, b):
    f32 = jnp.float32
    h = jnp.dot(feat, W, preferred_element_type=f32)
    h_blk = jnp.dot(feat_blk, W, preferred_element_type=f32)
    el = jnp.sum(h * al, axis=1, keepdims=True)
    er_row = lax.dot_general(ar, h_blk, (((1,), (1,)), ((), ())),
                             preferred_element_type=f32)
    er_col = jnp.sum(h_blk * ar, axis=1, keepdims=True)

    elmax = jnp.max(el)
    t = elmax + er_col
    c = jnp.where(t >= 0.0, t, _SLOPE * t)
    f1 = jnp.exp(t - c)
    f2 = jnp.exp(_SLOPE * t - c)

    u1 = jnp.exp(el - elmax)
    u2 = jnp.exp(_SLOPE * (el - elmax))
    rhs1 = jnp.concatenate([h * u1, u1], axis=1).astype(jnp.bfloat16)
    rhs2 = jnp.concatenate([h * u2, u2], axis=1).astype(jnp.bfloat16)

    x = el + er_row
    m1f = jnp.where(x >= 0.0, cnt_f, 0.0)
    m1 = m1f.astype(jnp.bfloat16)
    m2 = (cnt_f - m1f).astype(jnp.bfloat16)

    dn = (((0,), (0,)), ((), ()))
    a1 = lax.dot_general(m1, rhs1, dn, preferred_element_type=f32)
    a2 = lax.dot_general(m2, rhs2, dn, preferred_element_type=f32)
    num = f1 * a1[:, :_HID] + f2 * a2[:, :_HID]
    den = f1 * a1[:, _HID:_HID + 1] + f2 * a2[:, _HID:_HID + 1]
    return jnp.tanh(num / den + b)


def _gat3_kernel(adj_ref, feat_ref, aw_ref, gat_W_ref, gat_al_ref,
                 gat_ar_ref, gat_b_ref, gm_W_ref, gm_al_ref, gm_ar_ref,
                 gm_b_ref, mg_ref, m0_ref, m1_ref):
    j = pl.program_id(0)
    feat = feat_ref[...]
    feat_blk = feat_ref[pl.ds(j * _BJ, _BJ), :]

    a0 = adj_ref[0, :, :]
    a1 = adj_ref[1, :, :]

    row = lax.broadcasted_iota(jnp.int32, (_N, _BJ), 0)
    col = lax.broadcasted_iota(jnp.int32, (_N, _BJ), 1) + j * _BJ
    eye = (row == col).astype(jnp.float32)

    w = jax.nn.softmax(aw_ref[...])
    a0_eff = jnp.where(w[0, 0] != 0.0, a0, 0)
    a1_eff = jnp.where(w[0, 1] != 0.0, a1, 0)
    cnt_m = jnp.where((a0_eff | a1_eff) != 0, 1.0, 0.0) + eye
    cnt_0 = jnp.where(a0 != 0, 1.0, 0.0) + eye
    cnt_1 = jnp.where(a1 != 0, 1.0, 0.0) + eye

    mg_ref[...] = _gat_one(cnt_m, feat, feat_blk, gat_W_ref[...],
                           gat_al_ref[...], gat_ar_ref[...], gat_b_ref[...])
    m0_ref[...] = _gat_one(cnt_0, feat, feat_blk, gm_W_ref[0],
                           gm_al_ref[0:1, :], gm_ar_ref[0:1, :],
                           gm_b_ref[0:1, :])
    m1_ref[...] = _gat_one(cnt_1, feat, feat_blk, gm_W_ref[1],
                           gm_al_ref[1:2, :], gm_ar_ref[1:2, :],
                           gm_b_ref[1:2, :])


def _combine_kernel(mg_ref, m0_ref, m1_ref, sem_W1_ref, sem_b1_ref,
                    sem_q_ref, ft_W_ref, ft_b_ref, out_ref):
    mg = mg_ref[...]
    m0 = m0_ref[...]
    m1 = m1_ref[...]
    sem_W1 = sem_W1_ref[...]
    sem_b1 = sem_b1_ref[...]
    sem_q = sem_q_ref[...]

    def wp(x):
        t = jnp.tanh(jnp.dot(x, sem_W1, preferred_element_type=jnp.float32)
                     + sem_b1)
        return jnp.dot(t, sem_q, preferred_element_type=jnp.float32)

    s0 = jnp.sum(wp(mg)) / _N
    s1 = jnp.sum(wp(m0)) / _N
    s2 = jnp.sum(wp(m1)) / _N
    smax = jnp.maximum(jnp.maximum(s0, s1), s2)
    e0 = jnp.exp(s0 - smax)
    e1 = jnp.exp(s1 - smax)
    e2 = jnp.exp(s2 - smax)
    tot = e0 + e1 + e2
    semantic = (e0 / tot) * mg + (e1 / tot) * m0 + (e2 / tot) * m1

    ft_W = ft_W_ref[...]
    fa = (jnp.dot(mg, ft_W[0:_HID, :], preferred_element_type=jnp.float32)
          + jnp.dot(semantic, ft_W[_HID:2 * _HID, :],
                    preferred_element_type=jnp.float32)
          + ft_b_ref[...])
    out_ref[...] = jnp.tanh(fa)


def kernel(adj_list, feat, attention_weights, gat_W, gat_al, gat_ar, gat_b,
           gm_W, gm_al, gm_ar, gm_b, sem_W1, sem_b1, sem_q, ft_W, ft_b):
    aw = attention_weights.reshape(1, _M)
    gat_al2 = gat_al.reshape(1, _HID)
    gat_ar2 = gat_ar.reshape(1, _HID)
    gat_b2 = gat_b.reshape(1, _HID)
    sem_b12 = sem_b1.reshape(1, _SEM_HID)
    sem_q2 = sem_q.reshape(_SEM_HID, 1)
    ft_b2 = ft_b.reshape(1, _HID)

    num_blocks = _N // _BJ
    full = lambda shape: pl.BlockSpec(shape, lambda j: (0,) * len(shape))
    mg, m0, m1 = pl.pallas_call(
        _gat3_kernel,
        grid=(num_blocks,),
        in_specs=[
            pl.BlockSpec((_M, _N, _BJ), lambda j: (0, 0, j)),
            full((_N, _HID)),
            full((1, _M)),
            full((_HID, _HID)),
            full((1, _HID)),
            full((1, _HID)),
            full((1, _HID)),
            full((_M, _HID, _HID)),
            full((_M, _HID)),
            full((_M, _HID)),
            full((_M, _HID)),
        ],
        out_specs=[
            pl.BlockSpec((_BJ, _HID), lambda j: (j, 0)),
            pl.BlockSpec((_BJ, _HID), lambda j: (j, 0)),
            pl.BlockSpec((_BJ, _HID), lambda j: (j, 0)),
        ],
        out_shape=[jax.ShapeDtypeStruct((_N, _HID), jnp.float32)] * 3,
    )(adj_list, feat, aw, gat_W, gat_al2, gat_ar2, gat_b2,
      gm_W, gm_al, gm_ar, gm_b)

    out = pl.pallas_call(
        _combine_kernel,
        out_shape=jax.ShapeDtypeStruct((_N, _HID), jnp.float32),
    )(mg, m0, m1, sem_W1, sem_b12, sem_q2, ft_W, ft_b2)
    return out


# speedup vs baseline: 1.5715x; 1.0319x over previous
import jax
import jax.numpy as jnp
from jax import lax
from jax.experimental import pallas as pl
from jax.experimental.pallas import tpu as pltpu

_N = 1024
_HID = 64
_M = 2
_SEM_HID = 128
_BI = 256
_NB = _N // _BI
_SLOPE = 0.2


def _fused_kernel(adj_ref, feat_ref, aw_ref, gat_W_ref, gat_al_ref,
                  gat_ar_ref, gat_b_ref, gm_W_ref, gm_al_ref, gm_ar_ref,
                  gm_b_ref, sem_W1_ref, sem_b1_ref, sem_q_ref, ft_W_ref,
                  ft_b_ref, out_ref, acc1m, acc2m, acc10, acc20, acc11,
                  acc21):
    f32 = jnp.float32
    j = pl.program_id(0)
    feat = feat_ref[...]
    feat_blk = feat_ref[pl.ds(j * _BI, _BI), :]

    a0 = adj_ref[0, :, :]
    a1 = adj_ref[1, :, :]

    row = lax.broadcasted_iota(jnp.int32, (_BI, _N), 0) + j * _BI
    col = lax.broadcasted_iota(jnp.int32, (_BI, _N), 1)
    eye = (row == col).astype(f32)

    w = jax.nn.softmax(aw_ref[...])
    a0_eff = jnp.where(w[0, 0] != 0.0, a0, 0)
    a1_eff = jnp.where(w[0, 1] != 0.0, a1, 0)
    cnt_m = jnp.where((a0_eff | a1_eff) != 0, 1.0, 0.0) + eye
    cnt_0 = jnp.where(a0 != 0, 1.0, 0.0) + eye
    cnt_1 = jnp.where(a1 != 0, 1.0, 0.0) + eye

    graphs = (
        (cnt_m, gat_W_ref[...], gat_al_ref[...], gat_ar_ref[...], acc1m,
         acc2m),
        (cnt_0, gm_W_ref[0], gm_al_ref[0:1, :], gm_ar_ref[0:1, :], acc10,
         acc20),
        (cnt_1, gm_W_ref[1], gm_al_ref[1:2, :], gm_ar_ref[1:2, :], acc11,
         acc21),
    )

    dn = (((0,), (0,)), ((), ()))
    for cnt, W, al, ar, acc1, acc2 in graphs:
        h = jnp.dot(feat, W, preferred_element_type=f32)
        h_blk = jnp.dot(feat_blk, W, preferred_element_type=f32)
        el = jnp.sum(h * al, axis=1, keepdims=True)
        elmax = jnp.max(el)
        el_blk = jnp.sum(h_blk * al, axis=1, keepdims=True)
        er_row = lax.dot_general(ar, h, (((1,), (1,)), ((), ())),
                                 preferred_element_type=f32)

        x = el_blk + er_row
        m1f = jnp.where(x >= 0.0, cnt, 0.0)
        m1 = m1f.astype(jnp.bfloat16)
        m2 = (cnt - m1f).astype(jnp.bfloat16)

        u1 = jnp.exp(el_blk - elmax)
        u2 = jnp.exp(_SLOPE * (el_blk - elmax))
        rhs1 = jnp.concatenate([h_blk * u1, u1], axis=1).astype(jnp.bfloat16)
        rhs2 = jnp.concatenate([h_blk * u2, u2], axis=1).astype(jnp.bfloat16)

        r1 = lax.dot_general(m1, rhs1, dn, preferred_element_type=f32)
        r2 = lax.dot_general(m2, rhs2, dn, preferred_element_type=f32)

        @pl.when(j == 0)
        def _():
            acc1[...] = r1
            acc2[...] = r2

        @pl.when(j > 0)
        def _():
            acc1[...] += r1
            acc2[...] += r2

    @pl.when(j == _NB - 1)
    def _():
        zs = []
        for cnt, W, al, ar, acc1, acc2 in graphs:
            h = jnp.dot(feat, W, preferred_element_type=f32)
            el = jnp.sum(h * al, axis=1, keepdims=True)
            elmax = jnp.max(el)
            er_col = jnp.sum(h * ar, axis=1, keepdims=True)
            t = elmax + er_col
            c = jnp.where(t >= 0.0, t, _SLOPE * t)
            f1 = jnp.exp(t - c)
            f2 = jnp.exp(_SLOPE * t - c)
            A1 = acc1[...]
            A2 = acc2[...]
            num = f1 * A1[:, :_HID] + f2 * A2[:, :_HID]
            den = f1 * A1[:, _HID:_HID + 1] + f2 * A2[:, _HID:_HID + 1]
            zs.append(num / den)
        bm = gat_b_ref[...]
        mg = jnp.tanh(zs[0] + bm)
        m0 = jnp.tanh(zs[1] + gm_b_ref[0:1, :])
        m1_ = jnp.tanh(zs[2] + gm_b_ref[1:2, :])

        sem_W1 = sem_W1_ref[...]
        sem_b1 = sem_b1_ref[...]
        sem_q = sem_q_ref[...]

        def wp(xv):
            tt = jnp.tanh(jnp.dot(xv, sem_W1, preferred_element_type=f32)
                          + sem_b1)
            return jnp.dot(tt, sem_q, preferred_element_type=f32)

        s0 = jnp.sum(wp(mg)) / _N
        s1 = jnp.sum(wp(m0)) / _N
        s2 = jnp.sum(wp(m1_)) / _N
        smax = jnp.maximum(jnp.maximum(s0, s1), s2)
        e0 = jnp.exp(s0 - smax)
        e1 = jnp.exp(s1 - smax)
        e2 = jnp.exp(s2 - smax)
        tot = e0 + e1 + e2
        semantic = (e0 / tot) * mg + (e1 / tot) * m0 + (e2 / tot) * m1_

        ft_W = ft_W_ref[...]
        fa = (jnp.dot(mg, ft_W[0:_HID, :], preferred_element_type=f32)
              + jnp.dot(semantic, ft_W[_HID:2 * _HID, :],
                        preferred_element_type=f32)
              + ft_b_ref[...])
        out_ref[...] = jnp.tanh(fa)


def kernel(adj_list, feat, attention_weights, gat_W, gat_al, gat_ar, gat_b,
           gm_W, gm_al, gm_ar, gm_b, sem_W1, sem_b1, sem_q, ft_W, ft_b):
    aw = attention_weights.reshape(1, _M)
    gat_al2 = gat_al.reshape(1, _HID)
    gat_ar2 = gat_ar.reshape(1, _HID)
    gat_b2 = gat_b.reshape(1, _HID)
    sem_b12 = sem_b1.reshape(1, _SEM_HID)
    sem_q2 = sem_q.reshape(_SEM_HID, 1)
    ft_b2 = ft_b.reshape(1, _HID)

    full = lambda shape: pl.BlockSpec(shape, lambda j: (0,) * len(shape))
    acc = pltpu.VMEM((_N, _HID + 1), jnp.float32)
    out = pl.pallas_call(
        _fused_kernel,
        grid=(_NB,),
        in_specs=[
            pl.BlockSpec((_M, _BI, _N), lambda j: (0, j, 0)),
            full((_N, _HID)),
            full((1, _M)),
            full((_HID, _HID)),
            full((1, _HID)),
            full((1, _HID)),
            full((1, _HID)),
            full((_M, _HID, _HID)),
            full((_M, _HID)),
            full((_M, _HID)),
            full((_M, _HID)),
            full((_HID, _SEM_HID)),
            full((1, _SEM_HID)),
            full((_SEM_HID, 1)),
            full((2 * _HID, _HID)),
            full((1, _HID)),
        ],
        out_specs=pl.BlockSpec((_N, _HID), lambda j: (0, 0)),
        out_shape=jax.ShapeDtypeStruct((_N, _HID), jnp.float32),
        scratch_shapes=[acc] * 6,
    )(adj_list, feat, aw, gat_W, gat_al2, gat_ar2, gat_b2,
      gm_W, gm_al, gm_ar, gm_b, sem_W1, sem_b12, sem_q2, ft_W, ft_b2)
    return out


# speedup vs baseline: 1.6015x; 1.0191x over previous
import jax
import jax.numpy as jnp
from jax import lax
from jax.experimental import pallas as pl
from jax.experimental.pallas import tpu as pltpu

_N = 1024
_HID = 64
_M = 2
_SEM_HID = 128
_BI = 256
_NB = _N // _BI
_SLOPE = 0.2


def _fused_kernel(adj_ref, feat_ref, aw_ref, gat_W_ref, gat_al_ref,
                  gat_ar_ref, gat_b_ref, gm_W_ref, gm_al_ref, gm_ar_ref,
                  gm_b_ref, sem_W1_ref, sem_b1_ref, sem_q_ref, ft_W_ref,
                  ft_b_ref, out_ref, h_s, el_s, erow_s, ecol_s, rhs1_s,
                  rhs2_s, acc1_s, acc2_s):
    f32 = jnp.float32
    j = pl.program_id(0)

    @pl.when(j == 0)
    def _():
        feat = feat_ref[...]
        params = ((gat_W_ref[...], gat_al_ref[...], gat_ar_ref[...]),
                  (gm_W_ref[0], gm_al_ref[0:1, :], gm_ar_ref[0:1, :]),
                  (gm_W_ref[1], gm_al_ref[1:2, :], gm_ar_ref[1:2, :]))
        for g, (W, al, ar) in enumerate(params):
            h = jnp.dot(feat, W, preferred_element_type=f32)
            el = jnp.sum(h * al, axis=1, keepdims=True)
            elmax = jnp.max(el)
            u1 = jnp.exp(el - elmax)
            u2 = jnp.exp(_SLOPE * (el - elmax))
            h_s[g] = h
            el_s[g] = el
            erow_s[g] = lax.dot_general(ar, h, (((1,), (1,)), ((), ())),
                                        preferred_element_type=f32)
            ecol_s[g] = jnp.sum(h * ar, axis=1, keepdims=True)
            rhs1_s[g] = jnp.concatenate([h * u1, u1],
                                        axis=1).astype(jnp.bfloat16)
            rhs2_s[g] = jnp.concatenate([h * u2, u2],
                                        axis=1).astype(jnp.bfloat16)

    a0f = adj_ref[0, :, :].astype(f32)
    a1f = adj_ref[1, :, :].astype(f32)
    w = jax.nn.softmax(aw_ref[...])
    mm = a0f * w[0:1, 0:1] + a1f * w[0:1, 1:2]
    cnt_m = jnp.where(mm != 0.0, 1.0, 0.0)

    dn = (((0,), (0,)), ((), ()))
    for g, cnt in ((0, cnt_m), (1, a0f), (2, a1f)):
        el_blk = el_s[g, pl.ds(j * _BI, _BI), :]
        x = el_blk + erow_s[g]
        m1f = jnp.where(x >= 0.0, cnt, 0.0)
        m1 = m1f.astype(jnp.bfloat16)
        m2 = (cnt - m1f).astype(jnp.bfloat16)
        rhs1 = rhs1_s[g, pl.ds(j * _BI, _BI), :]
        rhs2 = rhs2_s[g, pl.ds(j * _BI, _BI), :]
        r1 = lax.dot_general(m1, rhs1, dn, preferred_element_type=f32)
        r2 = lax.dot_general(m2, rhs2, dn, preferred_element_type=f32)

        @pl.when(j == 0)
        def _():
            acc1_s[g] = r1
            acc2_s[g] = r2

        @pl.when(j > 0)
        def _():
            acc1_s[g] += r1
            acc2_s[g] += r2

    @pl.when(j == _NB - 1)
    def _():
        zs = []
        for g in range(3):
            h = h_s[g]
            el = el_s[g]
            elmax = jnp.max(el)
            er_col = ecol_s[g]
            t = elmax + er_col
            c = jnp.where(t >= 0.0, t, _SLOPE * t)
            f1 = jnp.exp(t - c)
            f2 = jnp.exp(_SLOPE * t - c)
            xd = el + er_col
            ed = jnp.where(xd >= 0.0, xd, _SLOPE * xd)
            term = jnp.exp(ed - c)
            A1 = acc1_s[g]
            A2 = acc2_s[g]
            num = f1 * A1[:, :_HID] + f2 * A2[:, :_HID] + term * h
            den = (f1 * A1[:, _HID:_HID + 1] + f2 * A2[:, _HID:_HID + 1]
                   + term)
            zs.append(num / den)
        mg = jnp.tanh(zs[0] + gat_b_ref[...])
        m0 = jnp.tanh(zs[1] + gm_b_ref[0:1, :])
        m1_ = jnp.tanh(zs[2] + gm_b_ref[1:2, :])

        sem_W1 = sem_W1_ref[...]
        sem_b1 = sem_b1_ref[...]
        sem_q = sem_q_ref[...]

        def wp(xv):
            tt = jnp.tanh(jnp.dot(xv, sem_W1, preferred_element_type=f32)
                          + sem_b1)
            return jnp.dot(tt, sem_q, preferred_element_type=f32)

        s0 = jnp.sum(wp(mg)) / _N
        s1 = jnp.sum(wp(m0)) / _N
        s2 = jnp.sum(wp(m1_)) / _N
        smax = jnp.maximum(jnp.maximum(s0, s1), s2)
        e0 = jnp.exp(s0 - smax)
        e1 = jnp.exp(s1 - smax)
        e2 = jnp.exp(s2 - smax)
        tot = e0 + e1 + e2
        semantic = (e0 / tot) * mg + (e1 / tot) * m0 + (e2 / tot) * m1_

        ft_W = ft_W_ref[...]
        fa = (jnp.dot(mg, ft_W[0:_HID, :], preferred_element_type=f32)
              + jnp.dot(semantic, ft_W[_HID:2 * _HID, :],
                        preferred_element_type=f32)
              + ft_b_ref[...])
        out_ref[...] = jnp.tanh(fa)


def kernel(adj_list, feat, attention_weights, gat_W, gat_al, gat_ar, gat_b,
           gm_W, gm_al, gm_ar, gm_b, sem_W1, sem_b1, sem_q, ft_W, ft_b):
    aw = attention_weights.reshape(1, _M)
    gat_al2 = gat_al.reshape(1, _HID)
    gat_ar2 = gat_ar.reshape(1, _HID)
    gat_b2 = gat_b.reshape(1, _HID)
    sem_b12 = sem_b1.reshape(1, _SEM_HID)
    sem_q2 = sem_q.reshape(_SEM_HID, 1)
    ft_b2 = ft_b.reshape(1, _HID)

    full = lambda shape: pl.BlockSpec(shape, lambda j: (0,) * len(shape))
    out = pl.pallas_call(
        _fused_kernel,
        grid=(_NB,),
        in_specs=[
            pl.BlockSpec((_M, _BI, _N), lambda j: (0, j, 0)),
            full((_N, _HID)),
            full((1, _M)),
            full((_HID, _HID)),
            full((1, _HID)),
            full((1, _HID)),
            full((1, _HID)),
            full((_M, _HID, _HID)),
            full((_M, _HID)),
            full((_M, _HID)),
            full((_M, _HID)),
            full((_HID, _SEM_HID)),
            full((1, _SEM_HID)),
            full((_SEM_HID, 1)),
            full((2 * _HID, _HID)),
            full((1, _HID)),
        ],
        out_specs=pl.BlockSpec((_N, _HID), lambda j: (0, 0)),
        out_shape=jax.ShapeDtypeStruct((_N, _HID), jnp.float32),
        scratch_shapes=[
            pltpu.VMEM((3, _N, _HID), jnp.float32),
            pltpu.VMEM((3, _N, 1), jnp.float32),
            pltpu.VMEM((3, 1, _N), jnp.float32),
            pltpu.VMEM((3, _N, 1), jnp.float32),
            pltpu.VMEM((3, _N, _HID + 1), jnp.bfloat16),
            pltpu.VMEM((3, _N, _HID + 1), jnp.bfloat16),
            pltpu.VMEM((3, _N, _HID + 1), jnp.float32),
            pltpu.VMEM((3, _N, _HID + 1), jnp.float32),
        ],
    )(adj_list, feat, aw, gat_W, gat_al2, gat_ar2, gat_b2,
      gm_W, gm_al, gm_ar, gm_b, sem_W1, sem_b12, sem_q2, ft_W, ft_b2)
    return out


# speedup vs baseline: 1.7508x; 1.0932x over previous
import jax
import jax.numpy as jnp
from jax import lax
from jax.experimental import pallas as pl
from jax.experimental.pallas import tpu as pltpu

_N = 1024
_HID = 64
_M = 2
_SEM_HID = 128
_BI = 256
_NB = _N // _BI
_SLOPE = 0.2


def _fused_kernel(adj_hbm, feat_ref, aw_ref, gat_W_ref, gat_al_ref,
                  gat_ar_ref, gat_b_ref, gm_W_ref, gm_al_ref, gm_ar_ref,
                  gm_b_ref, sem_W1_ref, sem_b1_ref, sem_q_ref, ft_W_ref,
                  ft_b_ref, out_ref, abuf, h_s, el_s, erow_s, ecol_s,
                  rhs1_s, rhs2_s, acc1_s, acc2_s, sem):
    f32 = jnp.float32
    j = pl.program_id(0)

    def copy(blk):
        return pltpu.make_async_copy(
            adj_hbm.at[:, pl.ds(blk * _BI, _BI), :],
            abuf.at[blk % 2], sem.at[blk % 2])

    @pl.when(j == 0)
    def _():
        copy(0).start()

    @pl.when(j + 1 < _NB)
    def _():
        copy(j + 1).start()

    @pl.when(j == 0)
    def _():
        feat = feat_ref[...]
        params = ((gat_W_ref[...], gat_al_ref[...].reshape(1, _HID),
                   gat_ar_ref[...].reshape(1, _HID)),
                  (gm_W_ref[0], gm_al_ref[0:1, :], gm_ar_ref[0:1, :]),
                  (gm_W_ref[1], gm_al_ref[1:2, :], gm_ar_ref[1:2, :]))
        for g, (W, al, ar) in enumerate(params):
            h = jnp.dot(feat, W, preferred_element_type=f32)
            el = jnp.sum(h * al, axis=1, keepdims=True)
            elmax = jnp.max(el)
            u1 = jnp.exp(el - elmax)
            u2 = jnp.exp(_SLOPE * (el - elmax))
            h_s[g] = h
            el_s[g] = el
            erow_s[g] = lax.dot_general(ar, h, (((1,), (1,)), ((), ())),
                                        preferred_element_type=f32)
            ecol_s[g] = jnp.sum(h * ar, axis=1, keepdims=True)
            rhs1_s[g] = jnp.concatenate([h * u1, u1],
                                        axis=1).astype(jnp.bfloat16)
            rhs2_s[g] = jnp.concatenate([h * u2, u2],
                                        axis=1).astype(jnp.bfloat16)

    copy(j).wait()

    a0f = abuf[j % 2, 0, :, :].astype(f32)
    a1f = abuf[j % 2, 1, :, :].astype(f32)
    w = jax.nn.softmax(aw_ref[...].reshape(1, _M))
    mm = a0f * w[0:1, 0:1] + a1f * w[0:1, 1:2]
    cnt_m = jnp.where(mm != 0.0, 1.0, 0.0)

    dn = (((0,), (0,)), ((), ()))
    for g, cnt in ((0, cnt_m), (1, a0f), (2, a1f)):
        el_blk = el_s[g, pl.ds(j * _BI, _BI), :]
        x = el_blk + erow_s[g]
        m1f = jnp.where(x >= 0.0, cnt, 0.0)
        m1 = m1f.astype(jnp.bfloat16)
        m2 = (cnt - m1f).astype(jnp.bfloat16)
        rhs1 = rhs1_s[g, pl.ds(j * _BI, _BI), :]
        rhs2 = rhs2_s[g, pl.ds(j * _BI, _BI), :]
        r1 = lax.dot_general(m1, rhs1, dn, preferred_element_type=f32)
        r2 = lax.dot_general(m2, rhs2, dn, preferred_element_type=f32)

        @pl.when(j == 0)
        def _():
            acc1_s[g] = r1
            acc2_s[g] = r2

        @pl.when(j > 0)
        def _():
            acc1_s[g] += r1
            acc2_s[g] += r2

    @pl.when(j == _NB - 1)
    def _():
        zs = []
        for g in range(3):
            h = h_s[g]
            el = el_s[g]
            elmax = jnp.max(el)
            er_col = ecol_s[g]
            t = elmax + er_col
            c = jnp.where(t >= 0.0, t, _SLOPE * t)
            f1 = jnp.exp(t - c)
            f2 = jnp.exp(_SLOPE * t - c)
            xd = el + er_col
            ed = jnp.where(xd >= 0.0, xd, _SLOPE * xd)
            term = jnp.exp(ed - c)
            A1 = acc1_s[g]
            A2 = acc2_s[g]
            num = f1 * A1[:, :_HID] + f2 * A2[:, :_HID] + term * h
            den = (f1 * A1[:, _HID:_HID + 1] + f2 * A2[:, _HID:_HID + 1]
                   + term)
            zs.append(num / den)
        mg = jnp.tanh(zs[0] + gat_b_ref[...].reshape(1, _HID))
        m0 = jnp.tanh(zs[1] + gm_b_ref[0:1, :])
        m1_ = jnp.tanh(zs[2] + gm_b_ref[1:2, :])

        sem_W1 = sem_W1_ref[...]
        sem_b1 = sem_b1_ref[...].reshape(1, _SEM_HID)
        sem_q = sem_q_ref[...].reshape(1, _SEM_HID)

        def wp(xv):
            tt = jnp.tanh(jnp.dot(xv, sem_W1, preferred_element_type=f32)
                          + sem_b1)
            return jnp.sum(tt * sem_q)

        s0 = wp(mg) / _N
        s1 = wp(m0) / _N
        s2 = wp(m1_) / _N
        smax = jnp.maximum(jnp.maximum(s0, s1), s2)
        e0 = jnp.exp(s0 - smax)
        e1 = jnp.exp(s1 - smax)
        e2 = jnp.exp(s2 - smax)
        tot = e0 + e1 + e2
        semantic = (e0 / tot) * mg + (e1 / tot) * m0 + (e2 / tot) * m1_

        ft_W = ft_W_ref[...]
        fa = (jnp.dot(mg, ft_W[0:_HID, :], preferred_element_type=f32)
              + jnp.dot(semantic, ft_W[_HID:2 * _HID, :],
                        preferred_element_type=f32)
              + ft_b_ref[...].reshape(1, _HID))
        out_ref[...] = jnp.tanh(fa)


def kernel(adj_list, feat, attention_weights, gat_W, gat_al, gat_ar, gat_b,
           gm_W, gm_al, gm_ar, gm_b, sem_W1, sem_b1, sem_q, ft_W, ft_b):
    full = lambda shape: pl.BlockSpec(shape, lambda j: (0,) * len(shape))
    out = pl.pallas_call(
        _fused_kernel,
        grid=(_NB,),
        in_specs=[
            pl.BlockSpec(memory_space=pl.ANY),
            full((_N, _HID)),
            full((_M,)),
            full((_HID, _HID)),
            full((_HID,)),
            full((_HID,)),
            full((_HID,)),
            full((_M, _HID, _HID)),
            full((_M, _HID)),
            full((_M, _HID)),
            full((_M, _HID)),
            full((_HID, _SEM_HID)),
            full((_SEM_HID,)),
            full((_SEM_HID,)),
            full((2 * _HID, _HID)),
            full((_HID,)),
        ],
        out_specs=pl.BlockSpec((_N, _HID), lambda j: (0, 0)),
        out_shape=jax.ShapeDtypeStruct((_N, _HID), jnp.float32),
        scratch_shapes=[
            pltpu.VMEM((2, _M, _BI, _N), jnp.int32),
            pltpu.VMEM((3, _N, _HID), jnp.float32),
            pltpu.VMEM((3, _N, 1), jnp.float32),
            pltpu.VMEM((3, 1, _N), jnp.float32),
            pltpu.VMEM((3, _N, 1), jnp.float32),
            pltpu.VMEM((3, _N, _HID + 1), jnp.bfloat16),
            pltpu.VMEM((3, _N, _HID + 1), jnp.bfloat16),
            pltpu.VMEM((3, _N, _HID + 1), jnp.float32),
            pltpu.VMEM((3, _N, _HID + 1), jnp.float32),
            pltpu.SemaphoreType.DMA((2,)),
        ],
    )(adj_list, feat, attention_weights, gat_W, gat_al, gat_ar, gat_b,
      gm_W, gm_al, gm_ar, gm_b, sem_W1, sem_b1, sem_q, ft_W, ft_b)
    return out
